# baseline (speedup 1.0000x reference)
import functools

import numpy as np
import jax
import jax.numpy as jnp
from jax import lax
from jax.experimental import pallas as pl
from jax.experimental.pallas import tpu as pltpu
from jax.experimental.pallas import tpu_sc as plsc

VOCAB = 100000
D = 128
B = 16
TX = 512
TY = 2048
T_OUT = TX + TY

NC = 2
NS = 16
NW = NC * NS
ROWS_W = (B * TY) // NW
CHUNK = 128
NCHUNK = ROWS_W // CHUNK
XROWS_W = (B * TX) // NW
LANES = 16


def _sine_pe(length, dim):
    pos = np.arange(length, dtype=np.float32)[:, None]
    div = np.exp(np.arange(0, dim, 2, dtype=np.float32) * -(np.log(10000.0) / dim))
    pe = np.zeros((length, dim), dtype=np.float32)
    pe[:, 0::2] = np.sin(pos * div)
    pe[:, 1::2] = np.cos(pos * div)
    return pe


_PE = _sine_pe(TY, D)

_mesh = plsc.VectorSubcoreMesh(core_axis_name="c", subcore_axis_name="s")


@functools.partial(
    pl.kernel,
    out_type=jax.ShapeDtypeStruct((B * T_OUT, D), jnp.float32),
    mesh=_mesh,
    scratch_types=[
        pltpu.VMEM((NCHUNK, CHUNK), jnp.int32),
        pltpu.VMEM((CHUNK, D), jnp.float32),
        pltpu.VMEM((CHUNK, D), jnp.float32),
        pltpu.VMEM((XROWS_W, D), jnp.float32),
        pltpu.SemaphoreType.DMA,
        pltpu.SemaphoreType.DMA,
    ],
)
def _emb_kernel(x_hbm, y_hbm, table_hbm, ape_hbm, out_hbm,
                idx_v, rows_v, pe_v, x_v, gsem, psem):
    wid = lax.axis_index("s") * NC + lax.axis_index("c")
    b = wid // 2
    half = wid % 2

    pltpu.sync_copy(y_hbm.at[pl.ds(wid * NCHUNK, NCHUNK)], idx_v)

    xsrc = b * TX + half * XROWS_W
    xdst = b * T_OUT + half * XROWS_W
    pltpu.sync_copy(x_hbm.at[pl.ds(xsrc, XROWS_W)], x_v)
    pltpu.sync_copy(x_v, out_hbm.at[pl.ds(xdst, XROWS_W)])

    obase = b * T_OUT + TX + half * (TY // 2)
    pbase = half * (TY // 2)

    for c in range(NCHUNK):
        gcp = pltpu.make_async_copy(table_hbm.at[idx_v.at[c]], rows_v, gsem)
        gcp.start()
        pcp = pltpu.make_async_copy(
            ape_hbm.at[pl.ds(pbase + c * CHUNK, CHUNK)], pe_v, psem)
        pcp.start()
        gcp.wait()
        pcp.wait()

        def add_body(r, carry):
            for j in range(D // LANES):
                sl = pl.ds(j * LANES, LANES)
                plsc.addupdate(rows_v.at[r, sl], pe_v[r, sl])
            return carry

        lax.fori_loop(0, CHUNK, add_body, 0)

        pltpu.sync_copy(rows_v, out_hbm.at[pl.ds(obase + c * CHUNK, CHUNK)])


def kernel(x, y, table, alpha):
    ape = alpha * jnp.asarray(_PE)
    y2 = y.astype(jnp.int32).reshape(NW * NCHUNK, CHUNK)
    x2 = x.reshape(B * TX, D)
    out = _emb_kernel(x2, y2, table, ape)
    return out.reshape(B, T_OUT, D)


# speedup vs baseline: 1.1898x; 1.1898x over previous
import functools

import numpy as np
import jax
import jax.numpy as jnp
from jax import lax
from jax.experimental import pallas as pl
from jax.experimental.pallas import tpu as pltpu
from jax.experimental.pallas import tpu_sc as plsc

VOCAB = 100000
D = 128
B = 16
TX = 512
TY = 2048
T_OUT = TX + TY

NC = 2
NS = 16
NW = NC * NS
ROWS_W = (B * TY) // NW
CHUNK = 128
NCHUNK = ROWS_W // CHUNK
XROWS_W = (B * TX) // NW
LANES = 16


def _sine_pe(length, dim):
    pos = np.arange(length, dtype=np.float32)[:, None]
    div = np.exp(np.arange(0, dim, 2, dtype=np.float32) * -(np.log(10000.0) / dim))
    pe = np.zeros((length, dim), dtype=np.float32)
    pe[:, 0::2] = np.sin(pos * div)
    pe[:, 1::2] = np.cos(pos * div)
    return pe


_PE = _sine_pe(TY, D)

_mesh = plsc.VectorSubcoreMesh(core_axis_name="c", subcore_axis_name="s")


@functools.partial(
    pl.kernel,
    out_type=jax.ShapeDtypeStruct((B, T_OUT, D), jnp.float32),
    mesh=_mesh,
    scratch_types=[
        pltpu.VMEM((NCHUNK, CHUNK), jnp.int32),
        pltpu.VMEM((2, CHUNK, D), jnp.float32),
        pltpu.VMEM((2, CHUNK, D), jnp.float32),
        pltpu.VMEM((XROWS_W, D), jnp.float32),
        pltpu.VMEM((LANES,), jnp.float32),
        pltpu.SemaphoreType.DMA,
        pltpu.SemaphoreType.DMA,
        pltpu.SemaphoreType.DMA,
        pltpu.SemaphoreType.DMA,
        pltpu.SemaphoreType.DMA,
        pltpu.SemaphoreType.DMA,
        pltpu.SemaphoreType.DMA,
        pltpu.SemaphoreType.DMA,
    ],
)
def _emb_kernel(x_hbm, y_hbm, table_hbm, pe_hbm, alpha_hbm, out_hbm,
                idx_v, rows_v, pe_v, x_v, alpha_v,
                gsem0, gsem1, psem0, psem1, osem0, osem1, xsem, isem):
    s = lax.axis_index("s")
    c = lax.axis_index("c")
    b = s
    half = c
    t0 = half * (TY // 2)

    gsems = (gsem0, gsem1)
    psems = (psem0, psem1)
    osems = (osem0, osem1)

    icp = pltpu.make_async_copy(
        y_hbm.at[pl.ds((b * NC + half) * NCHUNK, NCHUNK)], idx_v, isem)
    icp.start()
    xin = pltpu.make_async_copy(
        x_hbm.at[b, pl.ds(half * XROWS_W, XROWS_W)], x_v, xsem)
    xin.start()
    pltpu.sync_copy(alpha_hbm, alpha_v)
    aval = alpha_v[...]

    def gather_start(cch, slot):
        cp = pltpu.make_async_copy(
            table_hbm.at[idx_v.at[cch]], rows_v.at[slot], gsems[slot])
        cp.start()
        return cp

    def pe_start(cch, slot):
        cp = pltpu.make_async_copy(
            pe_hbm.at[pl.ds(t0 + cch * CHUNK, CHUNK)], pe_v.at[slot],
            psems[slot])
        cp.start()
        return cp

    def out_start(cch, slot):
        cp = pltpu.make_async_copy(
            rows_v.at[slot],
            out_hbm.at[b, pl.ds(TX + t0 + cch * CHUNK, CHUNK)], osems[slot])
        cp.start()
        return cp

    icp.wait()
    g_cp = gather_start(0, 0)
    p_cp = pe_start(0, 0)
    o_cps = [None, None]

    xin.wait()
    xout = pltpu.make_async_copy(
        x_v, out_hbm.at[b, pl.ds(half * XROWS_W, XROWS_W)], xsem)
    xout.start()

    for cch in range(NCHUNK):
        slot = cch % 2
        nslot = 1 - slot
        if cch + 1 < NCHUNK:
            if o_cps[nslot] is not None:
                o_cps[nslot].wait()
                o_cps[nslot] = None
            g_next = gather_start(cch + 1, nslot)
            p_next = pe_start(cch + 1, nslot)
        g_cp.wait()
        p_cp.wait()

        def add_body(r, carry):
            for j in range(D // LANES):
                sl = pl.ds(j * LANES, LANES)
                plsc.addupdate(rows_v.at[slot, r, sl], pe_v[slot, r, sl] * aval)
            return carry

        lax.fori_loop(0, CHUNK, add_body, 0)

        o_cps[slot] = out_start(cch, slot)
        if cch + 1 < NCHUNK:
            g_cp = g_next
            p_cp = p_next

    xout.wait()
    for cp in o_cps:
        if cp is not None:
            cp.wait()


def kernel(x, y, table, alpha):
    y2 = y.astype(jnp.int32).reshape(NW * NCHUNK, CHUNK)
    alpha_vec = jnp.broadcast_to(alpha.astype(jnp.float32), (LANES,))
    return _emb_kernel(x, y2, table, jnp.asarray(_PE), alpha_vec)


# speedup vs baseline: 1.2050x; 1.0128x over previous
import functools

import numpy as np
import jax
import jax.numpy as jnp
from jax import lax
from jax.experimental import pallas as pl
from jax.experimental.pallas import tpu as pltpu
from jax.experimental.pallas import tpu_sc as plsc

VOCAB = 100000
D = 128
B = 16
TX = 512
TY = 2048
T_OUT = TX + TY

NC = 2
NS = 16
NW = NC * NS
ROWS_W = (B * TY) // NW
CHUNK = 128
NCHUNK = ROWS_W // CHUNK
NSLOT = 4
XROWS_W = (B * TX) // NW


def _sine_pe(length, dim):
    pos = np.arange(length, dtype=np.float32)[:, None]
    div = np.exp(np.arange(0, dim, 2, dtype=np.float32) * -(np.log(10000.0) / dim))
    pe = np.zeros((length, dim), dtype=np.float32)
    pe[:, 0::2] = np.sin(pos * div)
    pe[:, 1::2] = np.cos(pos * div)
    return pe


_PE = _sine_pe(TY, D)

_mesh = plsc.VectorSubcoreMesh(core_axis_name="c", subcore_axis_name="s")


@functools.partial(
    pl.kernel,
    out_type=jax.ShapeDtypeStruct((B, T_OUT, D), jnp.float32),
    mesh=_mesh,
    scratch_types=[
        pltpu.VMEM((NCHUNK, CHUNK), jnp.int32),
        pltpu.VMEM((NSLOT, CHUNK, D), jnp.float32),
        pltpu.VMEM((XROWS_W, D), jnp.float32),
        [pltpu.SemaphoreType.DMA] * NSLOT,
        [pltpu.SemaphoreType.DMA] * NSLOT,
        [pltpu.SemaphoreType.DMA] * NSLOT,
        pltpu.SemaphoreType.DMA,
        pltpu.SemaphoreType.DMA,
    ],
)
def _emb_kernel(x_hbm, y_hbm, table_hbm, ape_hbm, out_hbm,
                idx_v, rows_v, x_v, psems, gsems, osems, xsem, isem):
    s = lax.axis_index("s")
    c = lax.axis_index("c")
    b = s
    half = c
    t0 = half * (TY // 2)

    icp = pltpu.make_async_copy(
        y_hbm.at[pl.ds((b * NC + half) * NCHUNK, NCHUNK)], idx_v, isem)
    icp.start()
    xin = pltpu.make_async_copy(
        x_hbm.at[b, pl.ds(half * XROWS_W, XROWS_W)], x_v, xsem)
    xin.start()

    def pe_start(cch, slot):
        return pltpu.async_copy(
            ape_hbm.at[pl.ds(t0 + cch * CHUNK, CHUNK)], rows_v.at[slot],
            psems[slot])

    def gather_start(cch, slot):
        return pltpu.async_copy(
            table_hbm.at[idx_v.at[cch]], rows_v.at[slot], gsems[slot],
            add=True)

    def out_start(cch, slot):
        return pltpu.async_copy(
            rows_v.at[slot],
            out_hbm.at[b, pl.ds(TX + t0 + cch * CHUNK, CHUNK)], osems[slot])

    icp.wait()

    xin.wait()
    xout = pltpu.make_async_copy(
        x_v, out_hbm.at[b, pl.ds(half * XROWS_W, XROWS_W)], xsem)
    xout.start()

    p_cps = [None] * NSLOT
    g_cps = [None] * NSLOT
    o_cps = [None] * NSLOT
    for step in range(NCHUNK + 2):
        c_p = step
        c_g = step - 1
        c_o = step - 2
        if c_p < NCHUNK:
            sp = c_p % NSLOT
            if o_cps[sp] is not None:
                o_cps[sp].wait()
                o_cps[sp] = None
            p_cps[sp] = pe_start(c_p, sp)
        if 0 <= c_g < NCHUNK:
            sg = c_g % NSLOT
            p_cps[sg].wait()
            g_cps[sg] = gather_start(c_g, sg)
        if 0 <= c_o < NCHUNK:
            so = c_o % NSLOT
            g_cps[so].wait()
            o_cps[so] = out_start(c_o, so)

    xout.wait()
    for cp in o_cps:
        if cp is not None:
            cp.wait()


def kernel(x, y, table, alpha):
    y2 = y.astype(jnp.int32).reshape(NW * NCHUNK, CHUNK)
    ape = alpha * jnp.asarray(_PE)
    return _emb_kernel(x, y2, table, ape)


# speedup vs baseline: 1.5892x; 1.3188x over previous
import functools

import numpy as np
import jax
import jax.numpy as jnp
from jax import lax
from jax.experimental import pallas as pl
from jax.experimental.pallas import tpu as pltpu
from jax.experimental.pallas import tpu_sc as plsc

VOCAB = 100000
D = 128
B = 16
TX = 512
TY = 2048
T_OUT = TX + TY

NC = 2
NS = 16
NW = NC * NS
ROWS_W = (B * TY) // NW
CHUNK = 128
NCHUNK = ROWS_W // CHUNK
NSLOT = 4
XROWS_W = (B * TX) // NW


def _sine_pe(length, dim):
    pos = np.arange(length, dtype=np.float32)[:, None]
    div = np.exp(np.arange(0, dim, 2, dtype=np.float32) * -(np.log(10000.0) / dim))
    pe = np.zeros((length, dim), dtype=np.float32)
    pe[:, 0::2] = np.sin(pos * div)
    pe[:, 1::2] = np.cos(pos * div)
    return pe


_PE = _sine_pe(TY, D)

_mesh = plsc.VectorSubcoreMesh(core_axis_name="c", subcore_axis_name="s")


@functools.partial(
    pl.kernel,
    out_type=jax.ShapeDtypeStruct((B, T_OUT, D), jnp.float32),
    mesh=_mesh,
    scratch_types=[
        pltpu.VMEM((NCHUNK, CHUNK), jnp.int32),
        pltpu.VMEM((NSLOT, CHUNK, D), jnp.float32),
        pltpu.VMEM((XROWS_W, D), jnp.float32),
        pltpu.VMEM_SHARED((TY // 2, D), jnp.float32),
        [pltpu.SemaphoreType.DMA] * NSLOT,
        [pltpu.SemaphoreType.DMA] * NSLOT,
        [pltpu.SemaphoreType.DMA] * NSLOT,
        pltpu.SemaphoreType.DMA,
        pltpu.SemaphoreType.DMA,
    ],
)
def _emb_kernel(x_hbm, y_hbm, table_hbm, ape_hbm, out_hbm,
                idx_v, rows_v, x_v, ape_sh, psems, gsems, osems, xsem, isem):
    s = lax.axis_index("s")
    c = lax.axis_index("c")
    b = s
    half = c
    t0 = half * (TY // 2)

    icp = pltpu.make_async_copy(
        y_hbm.at[pl.ds((b * NC + half) * NCHUNK, NCHUNK)], idx_v, isem)
    icp.start()
    xin = pltpu.make_async_copy(
        x_hbm.at[b, pl.ds(half * XROWS_W, XROWS_W)], x_v, xsem)
    xin.start()

    stage_rows = (TY // 2) // NS
    pltpu.sync_copy(
        ape_hbm.at[pl.ds(t0 + s * stage_rows, stage_rows)],
        ape_sh.at[pl.ds(s * stage_rows, stage_rows)])
    plsc.subcore_barrier()

    def pe_start(cch, slot):
        return pltpu.async_copy(
            ape_sh.at[pl.ds(cch * CHUNK, CHUNK)], rows_v.at[slot],
            psems[slot])

    def gather_start(cch, slot):
        return pltpu.async_copy(
            table_hbm.at[idx_v.at[cch]], rows_v.at[slot], gsems[slot],
            add=True)

    def out_start(cch, slot):
        return pltpu.async_copy(
            rows_v.at[slot],
            out_hbm.at[b, pl.ds(TX + t0 + cch * CHUNK, CHUNK)], osems[slot])

    icp.wait()

    xin.wait()
    xout = pltpu.make_async_copy(
        x_v, out_hbm.at[b, pl.ds(half * XROWS_W, XROWS_W)], xsem)
    xout.start()

    p_cps = [None] * NSLOT
    g_cps = [None] * NSLOT
    o_cps = [None] * NSLOT
    for step in range(NCHUNK + 2):
        c_p = step
        c_g = step - 1
        c_o = step - 2
        if c_p < NCHUNK:
            sp = c_p % NSLOT
            if o_cps[sp] is not None:
                o_cps[sp].wait()
                o_cps[sp] = None
            p_cps[sp] = pe_start(c_p, sp)
        if 0 <= c_g < NCHUNK:
            sg = c_g % NSLOT
            p_cps[sg].wait()
            g_cps[sg] = gather_start(c_g, sg)
        if 0 <= c_o < NCHUNK:
            so = c_o % NSLOT
            g_cps[so].wait()
            o_cps[so] = out_start(c_o, so)

    xout.wait()
    for cp in o_cps:
        if cp is not None:
            cp.wait()


def kernel(x, y, table, alpha):
    y2 = y.astype(jnp.int32).reshape(NW * NCHUNK, CHUNK)
    ape = alpha * jnp.asarray(_PE)
    return _emb_kernel(x, y2, table, ape)
